# Initial kernel scaffold; baseline (speedup 1.0000x reference)
import jax
import jax.numpy as jnp
from jax.experimental import pallas as pl


def kernel(input_ids, table, W_ih_0_fwd, W_hh_0_fwd, b_ih_0_fwd, b_hh_0_fwd, W_ih_0_bwd, W_hh_0_bwd, b_ih_0_bwd, b_hh_0_bwd, W_ih_1_fwd, W_hh_1_fwd, b_ih_1_fwd, b_hh_1_fwd, W_ih_1_bwd, W_hh_1_bwd, b_ih_1_bwd, b_hh_1_bwd, W_fc, b_fc):
    raise NotImplementedError("write your pallas kernel here")


# speedup vs baseline: 3.4703x; 3.4703x over previous
import functools

import jax
import jax.numpy as jnp
from jax import lax
from jax.experimental import pallas as pl
from jax.experimental.pallas import tpu as pltpu
from jax.experimental.pallas import tpu_sc as plsc

_B, _T = 1024, 200
_V, _E, _H, _C = 30522, 128, 256, 2
_N = _B * _T
_CH = 128


def _emb_gather(table, idx_flat):
    info = plsc.get_sparse_core_info()
    nc, ns = info.num_cores, info.num_subcores
    nw = nc * ns
    rows_per_w = _N // nw
    n_ch = rows_per_w // _CH
    idx3d = idx_flat.reshape(nw, n_ch, _CH)
    mesh = plsc.VectorSubcoreMesh(core_axis_name="c", subcore_axis_name="s")

    @functools.partial(
        pl.kernel,
        mesh=mesh,
        out_type=jax.ShapeDtypeStruct((_N, _E), jnp.float32),
        scratch_types=[
            pltpu.VMEM((n_ch, _CH), jnp.int32),
            pltpu.VMEM((_CH, _E), jnp.float32),
            pltpu.VMEM((_CH, _E), jnp.float32),
            pltpu.SemaphoreType.DMA,
            pltpu.SemaphoreType.DMA,
        ],
    )
    def gather_kernel(table_hbm, idx_hbm, out_hbm, idx_v, buf0, buf1, sem0, sem1):
        wid = lax.axis_index("s") * nc + lax.axis_index("c")
        base_row = wid * rows_per_w
        pltpu.sync_copy(idx_hbm.at[wid], idx_v)

        def body(j, carry):
            cp = pltpu.async_copy(table_hbm.at[idx_v.at[j]], buf0, sem0)
            cp.wait()
            pltpu.sync_copy(buf0, out_hbm.at[pl.ds(base_row + j * _CH, _CH)])
            return carry

        lax.fori_loop(0, n_ch, body, 0)

    return gather_kernel(table, idx3d)


def _lstm0_body(xf_ref, xb_ref, wxf, whf, bf, wxb, whb, bb,
                outf, outb, hf, cf, hb, cb):
    t = pl.program_id(0)

    @pl.when(t == 0)
    def _init():
        for r in (hf, cf, hb, cb):
            r[...] = jnp.zeros_like(r[...])

    def _step(x, wx, wh, b, h_s, c_s):
        gates = (jnp.dot(x, wx[...], preferred_element_type=jnp.float32)
                 + jnp.dot(h_s[...], wh[...], preferred_element_type=jnp.float32)
                 + b[...])
        i_g = jax.nn.sigmoid(gates[:, 0 * _H:1 * _H])
        f_g = jax.nn.sigmoid(gates[:, 1 * _H:2 * _H])
        g_g = jnp.tanh(gates[:, 2 * _H:3 * _H])
        o_g = jax.nn.sigmoid(gates[:, 3 * _H:4 * _H])
        c_new = f_g * c_s[...] + i_g * g_g
        h_new = o_g * jnp.tanh(c_new)
        h_s[...] = h_new
        c_s[...] = c_new
        return h_new

    outf[0] = _step(xf_ref[0], wxf, whf, bf, hf, cf)
    outb[0] = _step(xb_ref[0], wxb, whb, bb, hb, cb)


def _lstm1_body(oft, obt, ofr, obr, wxfa, wxfb, whf, bf,
                wxba, wxbb, whb, bb, wfc, bfc, out, hf, cf, hb, cb):
    t = pl.program_id(0)

    @pl.when(t == 0)
    def _init():
        for r in (hf, cf, hb, cb):
            r[...] = jnp.zeros_like(r[...])

    def _step(xa, xb, wxa, wxb, wh, b, h_s, c_s):
        gates = (jnp.dot(xa, wxa[...], preferred_element_type=jnp.float32)
                 + jnp.dot(xb, wxb[...], preferred_element_type=jnp.float32)
                 + jnp.dot(h_s[...], wh[...], preferred_element_type=jnp.float32)
                 + b[...])
        i_g = jax.nn.sigmoid(gates[:, 0 * _H:1 * _H])
        f_g = jax.nn.sigmoid(gates[:, 1 * _H:2 * _H])
        g_g = jnp.tanh(gates[:, 2 * _H:3 * _H])
        o_g = jax.nn.sigmoid(gates[:, 3 * _H:4 * _H])
        c_new = f_g * c_s[...] + i_g * g_g
        h_new = o_g * jnp.tanh(c_new)
        h_s[...] = h_new
        c_s[...] = c_new
        return h_new

    h_f = _step(oft[0], obt[0], wxfa, wxfb, whf, bf, hf, cf)
    h_b = _step(ofr[0], obr[0], wxba, wxbb, whb, bb, hb, cb)

    @pl.when(t == _T - 1)
    def _fc():
        w = wfc[...]
        out[...] = (jnp.dot(h_f, w[:_H], preferred_element_type=jnp.float32)
                    + jnp.dot(h_b, w[_H:], preferred_element_type=jnp.float32)
                    + bfc[...])


def _full_spec(shape):
    nd = len(shape)
    return pl.BlockSpec(shape, lambda t, _nd=nd: (0,) * _nd)


def _lstm_stack(x, p):
    f32 = jnp.float32

    wxf0 = p["W_ih_0_fwd"].T
    whf0 = p["W_hh_0_fwd"].T
    bf0 = (p["b_ih_0_fwd"] + p["b_hh_0_fwd"]).reshape(1, 4 * _H)
    wxb0 = p["W_ih_0_bwd"].T
    whb0 = p["W_hh_0_bwd"].T
    bb0 = (p["b_ih_0_bwd"] + p["b_hh_0_bwd"]).reshape(1, 4 * _H)

    outf0, outb0 = pl.pallas_call(
        _lstm0_body,
        grid=(_T,),
        in_specs=[
            pl.BlockSpec((1, _B, _E), lambda t: (t, 0, 0)),
            pl.BlockSpec((1, _B, _E), lambda t: (_T - 1 - t, 0, 0)),
            _full_spec((_E, 4 * _H)),
            _full_spec((_H, 4 * _H)),
            _full_spec((1, 4 * _H)),
            _full_spec((_E, 4 * _H)),
            _full_spec((_H, 4 * _H)),
            _full_spec((1, 4 * _H)),
        ],
        out_specs=[
            pl.BlockSpec((1, _B, _H), lambda t: (t, 0, 0)),
            pl.BlockSpec((1, _B, _H), lambda t: (_T - 1 - t, 0, 0)),
        ],
        out_shape=[
            jax.ShapeDtypeStruct((_T, _B, _H), f32),
            jax.ShapeDtypeStruct((_T, _B, _H), f32),
        ],
        scratch_shapes=[pltpu.VMEM((_B, _H), f32) for _ in range(4)],
        compiler_params=pltpu.CompilerParams(
            dimension_semantics=("arbitrary",)),
    )(x, x, wxf0, whf0, bf0, wxb0, whb0, bb0)

    wx1f = p["W_ih_1_fwd"].T
    wx1b = p["W_ih_1_bwd"].T
    wh1f = p["W_hh_1_fwd"].T
    wh1b = p["W_hh_1_bwd"].T
    b1f = (p["b_ih_1_fwd"] + p["b_hh_1_fwd"]).reshape(1, 4 * _H)
    b1b = (p["b_ih_1_bwd"] + p["b_hh_1_bwd"]).reshape(1, 4 * _H)
    wfc = p["W_fc"].T
    bfc = p["b_fc"].reshape(1, _C)

    logits = pl.pallas_call(
        _lstm1_body,
        grid=(_T,),
        in_specs=[
            pl.BlockSpec((1, _B, _H), lambda t: (t, 0, 0)),
            pl.BlockSpec((1, _B, _H), lambda t: (t, 0, 0)),
            pl.BlockSpec((1, _B, _H), lambda t: (_T - 1 - t, 0, 0)),
            pl.BlockSpec((1, _B, _H), lambda t: (_T - 1 - t, 0, 0)),
            _full_spec((_H, 4 * _H)),
            _full_spec((_H, 4 * _H)),
            _full_spec((_H, 4 * _H)),
            _full_spec((1, 4 * _H)),
            _full_spec((_H, 4 * _H)),
            _full_spec((_H, 4 * _H)),
            _full_spec((_H, 4 * _H)),
            _full_spec((1, 4 * _H)),
            _full_spec((2 * _H, _C)),
            _full_spec((1, _C)),
        ],
        out_specs=pl.BlockSpec((_B, _C), lambda t: (0, 0)),
        out_shape=jax.ShapeDtypeStruct((_B, _C), f32),
        scratch_shapes=[pltpu.VMEM((_B, _H), f32) for _ in range(4)],
        compiler_params=pltpu.CompilerParams(
            dimension_semantics=("arbitrary",)),
    )(outf0, outb0, outf0, outb0,
      wx1f[:_H], wx1f[_H:], wh1f, b1f,
      wx1b[:_H], wx1b[_H:], wh1b, b1b,
      wfc, bfc)
    return logits


def kernel(input_ids, table,
           W_ih_0_fwd, W_hh_0_fwd, b_ih_0_fwd, b_hh_0_fwd,
           W_ih_0_bwd, W_hh_0_bwd, b_ih_0_bwd, b_hh_0_bwd,
           W_ih_1_fwd, W_hh_1_fwd, b_ih_1_fwd, b_hh_1_fwd,
           W_ih_1_bwd, W_hh_1_bwd, b_ih_1_bwd, b_hh_1_bwd,
           W_fc, b_fc):
    p = dict(locals())
    input_ids = p.pop("input_ids")
    idx_flat = input_ids.T.reshape(_N).astype(jnp.int32)
    x_flat = _emb_gather(p["table"], idx_flat)
    x = x_flat.reshape(_T, _B, _E)
    return _lstm_stack(x, p)


# speedup vs baseline: 4.2094x; 1.2130x over previous
import functools

import jax
import jax.numpy as jnp
import numpy as np
from jax import lax
from jax.experimental import pallas as pl
from jax.experimental.pallas import tpu as pltpu
from jax.experimental.pallas import tpu_sc as plsc

_B, _T = 1024, 200
_V, _E, _H, _C = 30522, 128, 256, 2
_N = _B * _T
_CH = 128
_BF = jnp.bfloat16


def _emb_gather(table, idx_flat):
    info = plsc.get_sparse_core_info()
    nc, ns = info.num_cores, info.num_subcores
    nw = nc * ns
    rows_per_w = _N // nw
    n_ch = rows_per_w // _CH
    idx3d = idx_flat.reshape(nw, n_ch, _CH)
    mesh = plsc.VectorSubcoreMesh(core_axis_name="c", subcore_axis_name="s")

    @functools.partial(
        pl.kernel,
        mesh=mesh,
        out_type=jax.ShapeDtypeStruct((_N, _E), jnp.float32),
        scratch_types=[
            pltpu.VMEM((n_ch, _CH), jnp.int32),
            pltpu.VMEM((_CH, _E), jnp.float32),
            pltpu.VMEM((_CH, _E), jnp.float32),
            pltpu.SemaphoreType.DMA,
            pltpu.SemaphoreType.DMA,
        ],
    )
    def gather_kernel(table_hbm, idx_hbm, out_hbm, idx_v, buf0, buf1, sem0, sem1):
        wid = lax.axis_index("s") * nc + lax.axis_index("c")
        base_row = wid * rows_per_w
        pltpu.sync_copy(idx_hbm.at[wid], idx_v)

        def body(j, carry):
            cp = pltpu.async_copy(table_hbm.at[idx_v.at[j]], buf0, sem0)
            cp.wait()
            pltpu.sync_copy(buf0, out_hbm.at[pl.ds(base_row + j * _CH, _CH)])
            return carry

        lax.fori_loop(0, n_ch, body, 0)

    return gather_kernel(table, idx3d)


def _gate_step(xh_s, w, b, c_s, x_off):
    gates = jnp.dot(xh_s[...], w[...],
                    preferred_element_type=jnp.float32) + b[...]
    i_g = 0.5 * jnp.tanh(gates[:, 0 * _H:1 * _H]) + 0.5
    f_g = 0.5 * jnp.tanh(gates[:, 1 * _H:2 * _H]) + 0.5
    g_g = jnp.tanh(gates[:, 2 * _H:3 * _H])
    o_g = 0.5 * jnp.tanh(gates[:, 3 * _H:4 * _H]) + 0.5
    c_new = f_g * c_s[...] + i_g * g_g
    h_new = o_g * jnp.tanh(c_new)
    c_s[...] = c_new
    h_bf = h_new.astype(_BF)
    xh_s[:, x_off:] = h_bf
    return h_new, h_bf


def _lstm0_body(xf_ref, xb_ref, wf, bf, wb, bb,
                outf, outb, xhf, cf, xhb, cb):
    t = pl.program_id(0)

    @pl.when(t == 0)
    def _init():
        cf[...] = jnp.zeros_like(cf[...])
        cb[...] = jnp.zeros_like(cb[...])
        xhf[:, _E:] = jnp.zeros((_B, _H), _BF)
        xhb[:, _E:] = jnp.zeros((_B, _H), _BF)

    xhf[:, :_E] = xf_ref[0].astype(_BF)
    xhb[:, :_E] = xb_ref[0].astype(_BF)
    _, hf_bf = _gate_step(xhf, wf, bf, cf, _E)
    _, hb_bf = _gate_step(xhb, wb, bb, cb, _E)
    outf[0] = hf_bf
    outb[0] = hb_bf


def _lstm1_body(oft, obt, ofr, obr, wf, bf, wb, bb, wfc, bfc,
                out, xhf, cf, xhb, cb):
    t = pl.program_id(0)

    @pl.when(t == 0)
    def _init():
        cf[...] = jnp.zeros_like(cf[...])
        cb[...] = jnp.zeros_like(cb[...])
        xhf[:, 2 * _H:] = jnp.zeros((_B, _H), _BF)
        xhb[:, 2 * _H:] = jnp.zeros((_B, _H), _BF)

    xhf[:, 0 * _H:1 * _H] = oft[0]
    xhf[:, 1 * _H:2 * _H] = obt[0]
    xhb[:, 0 * _H:1 * _H] = ofr[0]
    xhb[:, 1 * _H:2 * _H] = obr[0]
    h_f, _ = _gate_step(xhf, wf, bf, cf, 2 * _H)
    h_b, _ = _gate_step(xhb, wb, bb, cb, 2 * _H)

    @pl.when(t == _T - 1)
    def _fc():
        w = wfc[...]
        out[...] = (jnp.dot(h_f, w[:_H], preferred_element_type=jnp.float32)
                    + jnp.dot(h_b, w[_H:], preferred_element_type=jnp.float32)
                    + bfc[...])


def _full_spec(shape):
    nd = len(shape)
    return pl.BlockSpec(shape, lambda t, _nd=nd: (0,) * _nd)


_GATE_SCALE = np.concatenate([
    np.full((_H,), 0.5, np.float32),
    np.full((_H,), 0.5, np.float32),
    np.ones((_H,), np.float32),
    np.full((_H,), 0.5, np.float32),
])


def _lstm_stack(x, p):
    f32 = jnp.float32

    wf0 = (jnp.concatenate([p["W_ih_0_fwd"].T, p["W_hh_0_fwd"].T])
           * _GATE_SCALE).astype(_BF)
    wb0 = (jnp.concatenate([p["W_ih_0_bwd"].T, p["W_hh_0_bwd"].T])
           * _GATE_SCALE).astype(_BF)
    bf0 = ((p["b_ih_0_fwd"] + p["b_hh_0_fwd"]) * _GATE_SCALE).reshape(1, 4 * _H)
    bb0 = ((p["b_ih_0_bwd"] + p["b_hh_0_bwd"]) * _GATE_SCALE).reshape(1, 4 * _H)

    outf0, outb0 = pl.pallas_call(
        _lstm0_body,
        grid=(_T,),
        in_specs=[
            pl.BlockSpec((1, _B, _E), lambda t: (t, 0, 0)),
            pl.BlockSpec((1, _B, _E), lambda t: (_T - 1 - t, 0, 0)),
            _full_spec((_E + _H, 4 * _H)),
            _full_spec((1, 4 * _H)),
            _full_spec((_E + _H, 4 * _H)),
            _full_spec((1, 4 * _H)),
        ],
        out_specs=[
            pl.BlockSpec((1, _B, _H), lambda t: (t, 0, 0)),
            pl.BlockSpec((1, _B, _H), lambda t: (_T - 1 - t, 0, 0)),
        ],
        out_shape=[
            jax.ShapeDtypeStruct((_T, _B, _H), _BF),
            jax.ShapeDtypeStruct((_T, _B, _H), _BF),
        ],
        scratch_shapes=[
            pltpu.VMEM((_B, _E + _H), _BF),
            pltpu.VMEM((_B, _H), f32),
            pltpu.VMEM((_B, _E + _H), _BF),
            pltpu.VMEM((_B, _H), f32),
        ],
        compiler_params=pltpu.CompilerParams(
            dimension_semantics=("arbitrary",)),
    )(x, x, wf0, bf0, wb0, bb0)

    w1f = (jnp.concatenate([p["W_ih_1_fwd"].T, p["W_hh_1_fwd"].T])
           * _GATE_SCALE).astype(_BF)
    w1b = (jnp.concatenate([p["W_ih_1_bwd"].T, p["W_hh_1_bwd"].T])
           * _GATE_SCALE).astype(_BF)
    b1f = ((p["b_ih_1_fwd"] + p["b_hh_1_fwd"]) * _GATE_SCALE).reshape(1, 4 * _H)
    b1b = ((p["b_ih_1_bwd"] + p["b_hh_1_bwd"]) * _GATE_SCALE).reshape(1, 4 * _H)
    wfc = p["W_fc"].T
    bfc = p["b_fc"].reshape(1, _C)

    logits = pl.pallas_call(
        _lstm1_body,
        grid=(_T,),
        in_specs=[
            pl.BlockSpec((1, _B, _H), lambda t: (t, 0, 0)),
            pl.BlockSpec((1, _B, _H), lambda t: (t, 0, 0)),
            pl.BlockSpec((1, _B, _H), lambda t: (_T - 1 - t, 0, 0)),
            pl.BlockSpec((1, _B, _H), lambda t: (_T - 1 - t, 0, 0)),
            _full_spec((3 * _H, 4 * _H)),
            _full_spec((1, 4 * _H)),
            _full_spec((3 * _H, 4 * _H)),
            _full_spec((1, 4 * _H)),
            _full_spec((2 * _H, _C)),
            _full_spec((1, _C)),
        ],
        out_specs=pl.BlockSpec((_B, _C), lambda t: (0, 0)),
        out_shape=jax.ShapeDtypeStruct((_B, _C), f32),
        scratch_shapes=[
            pltpu.VMEM((_B, 3 * _H), _BF),
            pltpu.VMEM((_B, _H), f32),
            pltpu.VMEM((_B, 3 * _H), _BF),
            pltpu.VMEM((_B, _H), f32),
        ],
        compiler_params=pltpu.CompilerParams(
            dimension_semantics=("arbitrary",)),
    )(outf0, outb0, outf0, outb0, w1f, b1f, w1b, b1b, wfc, bfc)
    return logits


def kernel(input_ids, table,
           W_ih_0_fwd, W_hh_0_fwd, b_ih_0_fwd, b_hh_0_fwd,
           W_ih_0_bwd, W_hh_0_bwd, b_ih_0_bwd, b_hh_0_bwd,
           W_ih_1_fwd, W_hh_1_fwd, b_ih_1_fwd, b_hh_1_fwd,
           W_ih_1_bwd, W_hh_1_bwd, b_ih_1_bwd, b_hh_1_bwd,
           W_fc, b_fc):
    p = dict(locals())
    input_ids = p.pop("input_ids")
    idx_flat = input_ids.T.reshape(_N).astype(jnp.int32)
    x_flat = _emb_gather(p["table"], idx_flat)
    x = x_flat.reshape(_T, _B, _E)
    return _lstm_stack(x, p)


# speedup vs baseline: 4.4326x; 1.0530x over previous
import functools

import jax
import jax.numpy as jnp
import numpy as np
from jax import lax
from jax.experimental import pallas as pl
from jax.experimental.pallas import tpu as pltpu
from jax.experimental.pallas import tpu_sc as plsc

_B, _T = 1024, 200
_V, _E, _H, _C = 30522, 128, 256, 2
_N = _B * _T
_CH = 128
_BF = jnp.bfloat16


def _emb_gather(table, idx_flat):
    info = plsc.get_sparse_core_info()
    nc, ns = info.num_cores, info.num_subcores
    nw = nc * ns
    rows_per_w = _N // nw
    n_ch = rows_per_w // _CH
    idx3d = idx_flat.reshape(nw, n_ch, _CH)
    mesh = plsc.VectorSubcoreMesh(core_axis_name="c", subcore_axis_name="s")

    @functools.partial(
        pl.kernel,
        mesh=mesh,
        out_type=jax.ShapeDtypeStruct((_N, _E), jnp.float32),
        scratch_types=[
            pltpu.VMEM((n_ch, _CH), jnp.int32),
            pltpu.VMEM((_CH, _E), jnp.float32),
            pltpu.VMEM((_CH, _E), jnp.float32),
            pltpu.SemaphoreType.DMA,
            pltpu.SemaphoreType.DMA,
        ],
    )
    def gather_kernel(table_hbm, idx_hbm, out_hbm, idx_v, buf0, buf1, sem0, sem1):
        wid = lax.axis_index("s") * nc + lax.axis_index("c")
        base_row = wid * rows_per_w
        pltpu.sync_copy(idx_hbm.at[wid], idx_v)

        def body(j, carry):
            cp = pltpu.async_copy(table_hbm.at[idx_v.at[j]], buf0, sem0)
            cp.wait()
            pltpu.sync_copy(buf0, out_hbm.at[pl.ds(base_row + j * _CH, _CH)])
            return carry

        lax.fori_loop(0, n_ch, body, 0)

    return gather_kernel(table, idx3d)


def _gate_step(xh_s, w, b, c_s, x_off):
    gates = jnp.dot(xh_s[...], w[...],
                    preferred_element_type=jnp.float32) + b[...]
    i_g = 0.5 * jnp.tanh(gates[:, 0 * _H:1 * _H]) + 0.5
    f_g = 0.5 * jnp.tanh(gates[:, 1 * _H:2 * _H]) + 0.5
    g_g = jnp.tanh(gates[:, 2 * _H:3 * _H])
    o_g = 0.5 * jnp.tanh(gates[:, 3 * _H:4 * _H]) + 0.5
    c_new = f_g * c_s[...] + i_g * g_g
    h_new = o_g * jnp.tanh(c_new)
    c_s[...] = c_new
    h_bf = h_new.astype(_BF)
    xh_s[:, x_off:] = h_bf
    return h_new, h_bf


def _lstm0_body(xf_ref, xb_ref, wf, bf, wb, bb,
                outf, outb, xhf, cf, xhb, cb):
    t = pl.program_id(0)

    @pl.when(t == 0)
    def _init():
        cf[...] = jnp.zeros_like(cf[...])
        cb[...] = jnp.zeros_like(cb[...])
        xhf[:, _E:] = jnp.zeros((_B, _H), _BF)
        xhb[:, _E:] = jnp.zeros((_B, _H), _BF)

    xhf[:, :_E] = xf_ref[0].astype(_BF)
    xhb[:, :_E] = xb_ref[1].astype(_BF)
    _, hf_bf = _gate_step(xhf, wf, bf, cf, _E)
    _, hb_bf = _gate_step(xhb, wb, bb, cb, _E)
    outf[0] = hf_bf
    outb[1] = hb_bf

    xhf[:, :_E] = xf_ref[1].astype(_BF)
    xhb[:, :_E] = xb_ref[0].astype(_BF)
    _, hf_bf = _gate_step(xhf, wf, bf, cf, _E)
    _, hb_bf = _gate_step(xhb, wb, bb, cb, _E)
    outf[1] = hf_bf
    outb[0] = hb_bf


def _lstm1_body(oft, obt, ofr, obr, wf, bf, wb, bb, wfc, bfc,
                out, xhf, cf, xhb, cb):
    t = pl.program_id(0)

    @pl.when(t == 0)
    def _init():
        cf[...] = jnp.zeros_like(cf[...])
        cb[...] = jnp.zeros_like(cb[...])
        xhf[:, 2 * _H:] = jnp.zeros((_B, _H), _BF)
        xhb[:, 2 * _H:] = jnp.zeros((_B, _H), _BF)

    xhf[:, 0 * _H:1 * _H] = oft[0]
    xhf[:, 1 * _H:2 * _H] = obt[0]
    xhb[:, 0 * _H:1 * _H] = ofr[1]
    xhb[:, 1 * _H:2 * _H] = obr[1]
    _gate_step(xhf, wf, bf, cf, 2 * _H)
    _gate_step(xhb, wb, bb, cb, 2 * _H)

    xhf[:, 0 * _H:1 * _H] = oft[1]
    xhf[:, 1 * _H:2 * _H] = obt[1]
    xhb[:, 0 * _H:1 * _H] = ofr[0]
    xhb[:, 1 * _H:2 * _H] = obr[0]
    h_f, _ = _gate_step(xhf, wf, bf, cf, 2 * _H)
    h_b, _ = _gate_step(xhb, wb, bb, cb, 2 * _H)

    @pl.when(t == _T // 2 - 1)
    def _fc():
        w = wfc[...]
        out[...] = (jnp.dot(h_f, w[:_H], preferred_element_type=jnp.float32)
                    + jnp.dot(h_b, w[_H:], preferred_element_type=jnp.float32)
                    + bfc[...])


def _full_spec(shape):
    nd = len(shape)
    return pl.BlockSpec(shape, lambda t, _nd=nd: (0,) * _nd)


_GATE_SCALE = np.concatenate([
    np.full((_H,), 0.5, np.float32),
    np.full((_H,), 0.5, np.float32),
    np.ones((_H,), np.float32),
    np.full((_H,), 0.5, np.float32),
])


def _lstm_stack(x, p):
    f32 = jnp.float32

    wf0 = (jnp.concatenate([p["W_ih_0_fwd"].T, p["W_hh_0_fwd"].T])
           * _GATE_SCALE).astype(_BF)
    wb0 = (jnp.concatenate([p["W_ih_0_bwd"].T, p["W_hh_0_bwd"].T])
           * _GATE_SCALE).astype(_BF)
    bf0 = ((p["b_ih_0_fwd"] + p["b_hh_0_fwd"]) * _GATE_SCALE).reshape(1, 4 * _H)
    bb0 = ((p["b_ih_0_bwd"] + p["b_hh_0_bwd"]) * _GATE_SCALE).reshape(1, 4 * _H)

    outf0, outb0 = pl.pallas_call(
        _lstm0_body,
        grid=(_T // 2,),
        in_specs=[
            pl.BlockSpec((2, _B, _E), lambda t: (t, 0, 0)),
            pl.BlockSpec((2, _B, _E), lambda t: (_T // 2 - 1 - t, 0, 0)),
            _full_spec((_E + _H, 4 * _H)),
            _full_spec((1, 4 * _H)),
            _full_spec((_E + _H, 4 * _H)),
            _full_spec((1, 4 * _H)),
        ],
        out_specs=[
            pl.BlockSpec((2, _B, _H), lambda t: (t, 0, 0)),
            pl.BlockSpec((2, _B, _H), lambda t: (_T // 2 - 1 - t, 0, 0)),
        ],
        out_shape=[
            jax.ShapeDtypeStruct((_T, _B, _H), _BF),
            jax.ShapeDtypeStruct((_T, _B, _H), _BF),
        ],
        scratch_shapes=[
            pltpu.VMEM((_B, _E + _H), _BF),
            pltpu.VMEM((_B, _H), f32),
            pltpu.VMEM((_B, _E + _H), _BF),
            pltpu.VMEM((_B, _H), f32),
        ],
        compiler_params=pltpu.CompilerParams(
            dimension_semantics=("arbitrary",)),
    )(x, x, wf0, bf0, wb0, bb0)

    w1f = (jnp.concatenate([p["W_ih_1_fwd"].T, p["W_hh_1_fwd"].T])
           * _GATE_SCALE).astype(_BF)
    w1b = (jnp.concatenate([p["W_ih_1_bwd"].T, p["W_hh_1_bwd"].T])
           * _GATE_SCALE).astype(_BF)
    b1f = ((p["b_ih_1_fwd"] + p["b_hh_1_fwd"]) * _GATE_SCALE).reshape(1, 4 * _H)
    b1b = ((p["b_ih_1_bwd"] + p["b_hh_1_bwd"]) * _GATE_SCALE).reshape(1, 4 * _H)
    wfc = p["W_fc"].T
    bfc = p["b_fc"].reshape(1, _C)

    logits = pl.pallas_call(
        _lstm1_body,
        grid=(_T // 2,),
        in_specs=[
            pl.BlockSpec((2, _B, _H), lambda t: (t, 0, 0)),
            pl.BlockSpec((2, _B, _H), lambda t: (t, 0, 0)),
            pl.BlockSpec((2, _B, _H), lambda t: (_T // 2 - 1 - t, 0, 0)),
            pl.BlockSpec((2, _B, _H), lambda t: (_T // 2 - 1 - t, 0, 0)),
            _full_spec((3 * _H, 4 * _H)),
            _full_spec((1, 4 * _H)),
            _full_spec((3 * _H, 4 * _H)),
            _full_spec((1, 4 * _H)),
            _full_spec((2 * _H, _C)),
            _full_spec((1, _C)),
        ],
        out_specs=pl.BlockSpec((_B, _C), lambda t: (0, 0)),
        out_shape=jax.ShapeDtypeStruct((_B, _C), f32),
        scratch_shapes=[
            pltpu.VMEM((_B, 3 * _H), _BF),
            pltpu.VMEM((_B, _H), f32),
            pltpu.VMEM((_B, 3 * _H), _BF),
            pltpu.VMEM((_B, _H), f32),
        ],
        compiler_params=pltpu.CompilerParams(
            dimension_semantics=("arbitrary",)),
    )(outf0, outb0, outf0, outb0, w1f, b1f, w1b, b1b, wfc, bfc)
    return logits


def kernel(input_ids, table,
           W_ih_0_fwd, W_hh_0_fwd, b_ih_0_fwd, b_hh_0_fwd,
           W_ih_0_bwd, W_hh_0_bwd, b_ih_0_bwd, b_hh_0_bwd,
           W_ih_1_fwd, W_hh_1_fwd, b_ih_1_fwd, b_hh_1_fwd,
           W_ih_1_bwd, W_hh_1_bwd, b_ih_1_bwd, b_hh_1_bwd,
           W_fc, b_fc):
    p = dict(locals())
    input_ids = p.pop("input_ids")
    idx_flat = input_ids.T.reshape(_N).astype(jnp.int32)
    x_flat = _emb_gather(p["table"], idx_flat)
    x = x_flat.reshape(_T, _B, _E)
    return _lstm_stack(x, p)


# speedup vs baseline: 4.5416x; 1.0246x over previous
import functools

import jax
import jax.numpy as jnp
import numpy as np
from jax import lax
from jax.experimental import pallas as pl
from jax.experimental.pallas import tpu as pltpu
from jax.experimental.pallas import tpu_sc as plsc

_B, _T = 1024, 200
_V, _E, _H, _C = 30522, 128, 256, 2
_N = _B * _T
_CH = 128
_BF = jnp.bfloat16


def _emb_gather(table, idx_flat):
    info = plsc.get_sparse_core_info()
    nc, ns = info.num_cores, info.num_subcores
    nw = nc * ns
    rows_per_w = _N // nw
    n_ch = rows_per_w // _CH
    idx3d = idx_flat.reshape(nw, n_ch, _CH)
    mesh = plsc.VectorSubcoreMesh(core_axis_name="c", subcore_axis_name="s")

    @functools.partial(
        pl.kernel,
        mesh=mesh,
        out_type=jax.ShapeDtypeStruct((_N, _E), jnp.float32),
        scratch_types=[
            pltpu.VMEM((n_ch, _CH), jnp.int32),
            pltpu.VMEM((_CH, _E), jnp.float32),
            pltpu.VMEM((_CH, _E), jnp.float32),
            pltpu.SemaphoreType.DMA,
            pltpu.SemaphoreType.DMA,
        ],
    )
    def gather_kernel(table_hbm, idx_hbm, out_hbm, idx_v, buf0, buf1, sem0, sem1):
        wid = lax.axis_index("s") * nc + lax.axis_index("c")
        base_row = wid * rows_per_w
        pltpu.sync_copy(idx_hbm.at[wid], idx_v)

        pltpu.async_copy(table_hbm.at[idx_v.at[0]], buf0, sem0)

        def body(j2, carry):
            j = 2 * j2
            pltpu.async_copy(table_hbm.at[idx_v.at[j + 1]], buf1, sem1)
            pltpu.make_async_copy(
                table_hbm.at[idx_v.at[j]], buf0, sem0).wait()
            pltpu.sync_copy(buf0, out_hbm.at[pl.ds(base_row + j * _CH, _CH)])

            @pl.when(j2 < n_ch // 2 - 1)
            def _fire_next():
                pltpu.async_copy(table_hbm.at[idx_v.at[j + 2]], buf0, sem0)

            pltpu.make_async_copy(
                table_hbm.at[idx_v.at[j + 1]], buf1, sem1).wait()
            pltpu.sync_copy(
                buf1, out_hbm.at[pl.ds(base_row + (j + 1) * _CH, _CH)])
            return carry

        lax.fori_loop(0, n_ch // 2, body, 0)

    return gather_kernel(table, idx3d)


def _gate_step(xh_s, w, b, c_s, x_off):
    gates = jnp.dot(xh_s[...], w[...],
                    preferred_element_type=jnp.float32) + b[...]
    i_g = 0.5 * jnp.tanh(gates[:, 0 * _H:1 * _H]) + 0.5
    f_g = 0.5 * jnp.tanh(gates[:, 1 * _H:2 * _H]) + 0.5
    g_g = jnp.tanh(gates[:, 2 * _H:3 * _H])
    o_g = 0.5 * jnp.tanh(gates[:, 3 * _H:4 * _H]) + 0.5
    c_new = f_g * c_s[...] + i_g * g_g
    h_new = o_g * jnp.tanh(c_new)
    c_s[...] = c_new
    h_bf = h_new.astype(_BF)
    xh_s[:, x_off:] = h_bf
    return h_new, h_bf


def _lstm0_body(xf_ref, xb_ref, wf, bf, wb, bb,
                outf, outb, xhf, cf, xhb, cb):
    t = pl.program_id(0)

    @pl.when(t == 0)
    def _init():
        cf[...] = jnp.zeros_like(cf[...])
        cb[...] = jnp.zeros_like(cb[...])
        xhf[:, _E:] = jnp.zeros((_B, _H), _BF)
        xhb[:, _E:] = jnp.zeros((_B, _H), _BF)

    xhf[:, :_E] = xf_ref[0].astype(_BF)
    xhb[:, :_E] = xb_ref[1].astype(_BF)
    _, hf_bf = _gate_step(xhf, wf, bf, cf, _E)
    _, hb_bf = _gate_step(xhb, wb, bb, cb, _E)
    outf[0] = hf_bf
    outb[1] = hb_bf

    xhf[:, :_E] = xf_ref[1].astype(_BF)
    xhb[:, :_E] = xb_ref[0].astype(_BF)
    _, hf_bf = _gate_step(xhf, wf, bf, cf, _E)
    _, hb_bf = _gate_step(xhb, wb, bb, cb, _E)
    outf[1] = hf_bf
    outb[0] = hb_bf


def _lstm1_body(oft, obt, ofr, obr, wf, bf, wb, bb, wfc, bfc,
                out, xhf, cf, xhb, cb):
    t = pl.program_id(0)

    @pl.when(t == 0)
    def _init():
        cf[...] = jnp.zeros_like(cf[...])
        cb[...] = jnp.zeros_like(cb[...])
        xhf[:, 2 * _H:] = jnp.zeros((_B, _H), _BF)
        xhb[:, 2 * _H:] = jnp.zeros((_B, _H), _BF)

    xhf[:, 0 * _H:1 * _H] = oft[0]
    xhf[:, 1 * _H:2 * _H] = obt[0]
    xhb[:, 0 * _H:1 * _H] = ofr[1]
    xhb[:, 1 * _H:2 * _H] = obr[1]
    _gate_step(xhf, wf, bf, cf, 2 * _H)
    _gate_step(xhb, wb, bb, cb, 2 * _H)

    xhf[:, 0 * _H:1 * _H] = oft[1]
    xhf[:, 1 * _H:2 * _H] = obt[1]
    xhb[:, 0 * _H:1 * _H] = ofr[0]
    xhb[:, 1 * _H:2 * _H] = obr[0]
    h_f, _ = _gate_step(xhf, wf, bf, cf, 2 * _H)
    h_b, _ = _gate_step(xhb, wb, bb, cb, 2 * _H)

    @pl.when(t == _T // 2 - 1)
    def _fc():
        w = wfc[...]
        out[...] = (jnp.dot(h_f, w[:_H], preferred_element_type=jnp.float32)
                    + jnp.dot(h_b, w[_H:], preferred_element_type=jnp.float32)
                    + bfc[...])


def _full_spec(shape):
    nd = len(shape)
    return pl.BlockSpec(shape, lambda t, _nd=nd: (0,) * _nd)


_GATE_SCALE = np.concatenate([
    np.full((_H,), 0.5, np.float32),
    np.full((_H,), 0.5, np.float32),
    np.ones((_H,), np.float32),
    np.full((_H,), 0.5, np.float32),
])


def _lstm_stack(x, p):
    f32 = jnp.float32

    wf0 = (jnp.concatenate([p["W_ih_0_fwd"].T, p["W_hh_0_fwd"].T])
           * _GATE_SCALE).astype(_BF)
    wb0 = (jnp.concatenate([p["W_ih_0_bwd"].T, p["W_hh_0_bwd"].T])
           * _GATE_SCALE).astype(_BF)
    bf0 = ((p["b_ih_0_fwd"] + p["b_hh_0_fwd"]) * _GATE_SCALE).reshape(1, 4 * _H)
    bb0 = ((p["b_ih_0_bwd"] + p["b_hh_0_bwd"]) * _GATE_SCALE).reshape(1, 4 * _H)

    outf0, outb0 = pl.pallas_call(
        _lstm0_body,
        grid=(_T // 2,),
        in_specs=[
            pl.BlockSpec((2, _B, _E), lambda t: (t, 0, 0)),
            pl.BlockSpec((2, _B, _E), lambda t: (_T // 2 - 1 - t, 0, 0)),
            _full_spec((_E + _H, 4 * _H)),
            _full_spec((1, 4 * _H)),
            _full_spec((_E + _H, 4 * _H)),
            _full_spec((1, 4 * _H)),
        ],
        out_specs=[
            pl.BlockSpec((2, _B, _H), lambda t: (t, 0, 0)),
            pl.BlockSpec((2, _B, _H), lambda t: (_T // 2 - 1 - t, 0, 0)),
        ],
        out_shape=[
            jax.ShapeDtypeStruct((_T, _B, _H), _BF),
            jax.ShapeDtypeStruct((_T, _B, _H), _BF),
        ],
        scratch_shapes=[
            pltpu.VMEM((_B, _E + _H), _BF),
            pltpu.VMEM((_B, _H), f32),
            pltpu.VMEM((_B, _E + _H), _BF),
            pltpu.VMEM((_B, _H), f32),
        ],
        compiler_params=pltpu.CompilerParams(
            dimension_semantics=("arbitrary",)),
    )(x, x, wf0, bf0, wb0, bb0)

    w1f = (jnp.concatenate([p["W_ih_1_fwd"].T, p["W_hh_1_fwd"].T])
           * _GATE_SCALE).astype(_BF)
    w1b = (jnp.concatenate([p["W_ih_1_bwd"].T, p["W_hh_1_bwd"].T])
           * _GATE_SCALE).astype(_BF)
    b1f = ((p["b_ih_1_fwd"] + p["b_hh_1_fwd"]) * _GATE_SCALE).reshape(1, 4 * _H)
    b1b = ((p["b_ih_1_bwd"] + p["b_hh_1_bwd"]) * _GATE_SCALE).reshape(1, 4 * _H)
    wfc = p["W_fc"].T
    bfc = p["b_fc"].reshape(1, _C)

    logits = pl.pallas_call(
        _lstm1_body,
        grid=(_T // 2,),
        in_specs=[
            pl.BlockSpec((2, _B, _H), lambda t: (t, 0, 0)),
            pl.BlockSpec((2, _B, _H), lambda t: (t, 0, 0)),
            pl.BlockSpec((2, _B, _H), lambda t: (_T // 2 - 1 - t, 0, 0)),
            pl.BlockSpec((2, _B, _H), lambda t: (_T // 2 - 1 - t, 0, 0)),
            _full_spec((3 * _H, 4 * _H)),
            _full_spec((1, 4 * _H)),
            _full_spec((3 * _H, 4 * _H)),
            _full_spec((1, 4 * _H)),
            _full_spec((2 * _H, _C)),
            _full_spec((1, _C)),
        ],
        out_specs=pl.BlockSpec((_B, _C), lambda t: (0, 0)),
        out_shape=jax.ShapeDtypeStruct((_B, _C), f32),
        scratch_shapes=[
            pltpu.VMEM((_B, 3 * _H), _BF),
            pltpu.VMEM((_B, _H), f32),
            pltpu.VMEM((_B, 3 * _H), _BF),
            pltpu.VMEM((_B, _H), f32),
        ],
        compiler_params=pltpu.CompilerParams(
            dimension_semantics=("arbitrary",)),
    )(outf0, outb0, outf0, outb0, w1f, b1f, w1b, b1b, wfc, bfc)
    return logits


def kernel(input_ids, table,
           W_ih_0_fwd, W_hh_0_fwd, b_ih_0_fwd, b_hh_0_fwd,
           W_ih_0_bwd, W_hh_0_bwd, b_ih_0_bwd, b_hh_0_bwd,
           W_ih_1_fwd, W_hh_1_fwd, b_ih_1_fwd, b_hh_1_fwd,
           W_ih_1_bwd, W_hh_1_bwd, b_ih_1_bwd, b_hh_1_bwd,
           W_fc, b_fc):
    p = dict(locals())
    input_ids = p.pop("input_ids")
    idx_flat = input_ids.T.reshape(_N).astype(jnp.int32)
    x_flat = _emb_gather(p["table"], idx_flat)
    x = x_flat.reshape(_T, _B, _E)
    return _lstm_stack(x, p)


# speedup vs baseline: 4.6650x; 1.0272x over previous
import functools

import jax
import jax.numpy as jnp
import numpy as np
from jax import lax
from jax.experimental import pallas as pl
from jax.experimental.pallas import tpu as pltpu
from jax.experimental.pallas import tpu_sc as plsc

_B, _T = 1024, 200
_V, _E, _H, _C = 30522, 128, 256, 2
_N = _B * _T
_S = 4
_CH = 128
_BF = jnp.bfloat16


def _emb_gather(table, idx_flat):
    info = plsc.get_sparse_core_info()
    nc, ns = info.num_cores, info.num_subcores
    nw = nc * ns
    rows_per_w = _N // nw
    n_ch = rows_per_w // _CH
    idx3d = idx_flat.reshape(nw, n_ch, _CH)
    mesh = plsc.VectorSubcoreMesh(core_axis_name="c", subcore_axis_name="s")

    @functools.partial(
        pl.kernel,
        mesh=mesh,
        out_type=jax.ShapeDtypeStruct((_N, _E), jnp.float32),
        scratch_types=[
            pltpu.VMEM((n_ch, _CH), jnp.int32),
            pltpu.VMEM((_CH, _E), jnp.float32),
            pltpu.VMEM((_CH, _E), jnp.float32),
            pltpu.SemaphoreType.DMA,
            pltpu.SemaphoreType.DMA,
        ],
    )
    def gather_kernel(table_hbm, idx_hbm, out_hbm, idx_v, buf0, buf1, sem0, sem1):
        wid = lax.axis_index("s") * nc + lax.axis_index("c")
        base_row = wid * rows_per_w
        pltpu.sync_copy(idx_hbm.at[wid], idx_v)

        pltpu.async_copy(table_hbm.at[idx_v.at[0]], buf0, sem0)

        def body(j2, carry):
            j = 2 * j2
            pltpu.async_copy(table_hbm.at[idx_v.at[j + 1]], buf1, sem1)
            pltpu.make_async_copy(
                table_hbm.at[idx_v.at[j]], buf0, sem0).wait()
            pltpu.sync_copy(buf0, out_hbm.at[pl.ds(base_row + j * _CH, _CH)])

            @pl.when(j2 < n_ch // 2 - 1)
            def _fire_next():
                pltpu.async_copy(table_hbm.at[idx_v.at[j + 2]], buf0, sem0)

            pltpu.make_async_copy(
                table_hbm.at[idx_v.at[j + 1]], buf1, sem1).wait()
            pltpu.sync_copy(
                buf1, out_hbm.at[pl.ds(base_row + (j + 1) * _CH, _CH)])
            return carry

        lax.fori_loop(0, n_ch // 2, body, 0)

    return gather_kernel(table, idx3d)


def _gate_step(xh_s, w, b, c_s, x_off):
    gates = jnp.dot(xh_s[...], w[...],
                    preferred_element_type=jnp.float32) + b[...]
    i_g = 0.5 * jnp.tanh(gates[:, 0 * _H:1 * _H]) + 0.5
    f_g = 0.5 * jnp.tanh(gates[:, 1 * _H:2 * _H]) + 0.5
    g_g = jnp.tanh(gates[:, 2 * _H:3 * _H])
    o_g = 0.5 * jnp.tanh(gates[:, 3 * _H:4 * _H]) + 0.5
    c_new = f_g * c_s[...] + i_g * g_g
    h_new = o_g * jnp.tanh(c_new)
    c_s[...] = c_new
    h_bf = h_new.astype(_BF)
    xh_s[:, x_off:] = h_bf
    return h_new, h_bf


def _lstm0_body(xf_ref, xb_ref, wf, bf, wb, bb,
                outf, outb, xhf, cf, xhb, cb):
    t = pl.program_id(0)

    @pl.when(t == 0)
    def _init():
        cf[...] = jnp.zeros_like(cf[...])
        cb[...] = jnp.zeros_like(cb[...])
        xhf[:, _E:] = jnp.zeros((_B, _H), _BF)
        xhb[:, _E:] = jnp.zeros((_B, _H), _BF)

    for s in range(_S):
        xhf[:, :_E] = xf_ref[s].astype(_BF)
        xhb[:, :_E] = xb_ref[_S - 1 - s].astype(_BF)
        _, hf_bf = _gate_step(xhf, wf, bf, cf, _E)
        _, hb_bf = _gate_step(xhb, wb, bb, cb, _E)
        outf[s] = hf_bf
        outb[_S - 1 - s] = hb_bf


def _lstm1_body(oft, obt, ofr, obr, wf, bf, wb, bb, wfc, bfc,
                out, xhf, cf, xhb, cb):
    t = pl.program_id(0)

    @pl.when(t == 0)
    def _init():
        cf[...] = jnp.zeros_like(cf[...])
        cb[...] = jnp.zeros_like(cb[...])
        xhf[:, 2 * _H:] = jnp.zeros((_B, _H), _BF)
        xhb[:, 2 * _H:] = jnp.zeros((_B, _H), _BF)

    for s in range(_S):
        xhf[:, 0 * _H:1 * _H] = oft[s]
        xhf[:, 1 * _H:2 * _H] = obt[s]
        xhb[:, 0 * _H:1 * _H] = ofr[_S - 1 - s]
        xhb[:, 1 * _H:2 * _H] = obr[_S - 1 - s]
        h_f, _ = _gate_step(xhf, wf, bf, cf, 2 * _H)
        h_b, _ = _gate_step(xhb, wb, bb, cb, 2 * _H)

    @pl.when(t == _T // _S - 1)
    def _fc():
        w = wfc[...]
        out[...] = (jnp.dot(h_f, w[:_H], preferred_element_type=jnp.float32)
                    + jnp.dot(h_b, w[_H:], preferred_element_type=jnp.float32)
                    + bfc[...])


def _full_spec(shape):
    nd = len(shape)
    return pl.BlockSpec(shape, lambda t, _nd=nd: (0,) * _nd)


_GATE_SCALE = np.concatenate([
    np.full((_H,), 0.5, np.float32),
    np.full((_H,), 0.5, np.float32),
    np.ones((_H,), np.float32),
    np.full((_H,), 0.5, np.float32),
])


def _lstm_stack(x, p):
    f32 = jnp.float32

    wf0 = (jnp.concatenate([p["W_ih_0_fwd"].T, p["W_hh_0_fwd"].T])
           * _GATE_SCALE).astype(_BF)
    wb0 = (jnp.concatenate([p["W_ih_0_bwd"].T, p["W_hh_0_bwd"].T])
           * _GATE_SCALE).astype(_BF)
    bf0 = ((p["b_ih_0_fwd"] + p["b_hh_0_fwd"]) * _GATE_SCALE).reshape(1, 4 * _H)
    bb0 = ((p["b_ih_0_bwd"] + p["b_hh_0_bwd"]) * _GATE_SCALE).reshape(1, 4 * _H)

    outf0, outb0 = pl.pallas_call(
        _lstm0_body,
        grid=(_T // _S,),
        in_specs=[
            pl.BlockSpec((_S, _B, _E), lambda t: (t, 0, 0)),
            pl.BlockSpec((_S, _B, _E), lambda t: (_T // _S - 1 - t, 0, 0)),
            _full_spec((_E + _H, 4 * _H)),
            _full_spec((1, 4 * _H)),
            _full_spec((_E + _H, 4 * _H)),
            _full_spec((1, 4 * _H)),
        ],
        out_specs=[
            pl.BlockSpec((_S, _B, _H), lambda t: (t, 0, 0)),
            pl.BlockSpec((_S, _B, _H), lambda t: (_T // _S - 1 - t, 0, 0)),
        ],
        out_shape=[
            jax.ShapeDtypeStruct((_T, _B, _H), _BF),
            jax.ShapeDtypeStruct((_T, _B, _H), _BF),
        ],
        scratch_shapes=[
            pltpu.VMEM((_B, _E + _H), _BF),
            pltpu.VMEM((_B, _H), f32),
            pltpu.VMEM((_B, _E + _H), _BF),
            pltpu.VMEM((_B, _H), f32),
        ],
        compiler_params=pltpu.CompilerParams(
            dimension_semantics=("arbitrary",)),
    )(x, x, wf0, bf0, wb0, bb0)

    w1f = (jnp.concatenate([p["W_ih_1_fwd"].T, p["W_hh_1_fwd"].T])
           * _GATE_SCALE).astype(_BF)
    w1b = (jnp.concatenate([p["W_ih_1_bwd"].T, p["W_hh_1_bwd"].T])
           * _GATE_SCALE).astype(_BF)
    b1f = ((p["b_ih_1_fwd"] + p["b_hh_1_fwd"]) * _GATE_SCALE).reshape(1, 4 * _H)
    b1b = ((p["b_ih_1_bwd"] + p["b_hh_1_bwd"]) * _GATE_SCALE).reshape(1, 4 * _H)
    wfc = p["W_fc"].T
    bfc = p["b_fc"].reshape(1, _C)

    logits = pl.pallas_call(
        _lstm1_body,
        grid=(_T // _S,),
        in_specs=[
            pl.BlockSpec((_S, _B, _H), lambda t: (t, 0, 0)),
            pl.BlockSpec((_S, _B, _H), lambda t: (t, 0, 0)),
            pl.BlockSpec((_S, _B, _H), lambda t: (_T // _S - 1 - t, 0, 0)),
            pl.BlockSpec((_S, _B, _H), lambda t: (_T // _S - 1 - t, 0, 0)),
            _full_spec((3 * _H, 4 * _H)),
            _full_spec((1, 4 * _H)),
            _full_spec((3 * _H, 4 * _H)),
            _full_spec((1, 4 * _H)),
            _full_spec((2 * _H, _C)),
            _full_spec((1, _C)),
        ],
        out_specs=pl.BlockSpec((_B, _C), lambda t: (0, 0)),
        out_shape=jax.ShapeDtypeStruct((_B, _C), f32),
        scratch_shapes=[
            pltpu.VMEM((_B, 3 * _H), _BF),
            pltpu.VMEM((_B, _H), f32),
            pltpu.VMEM((_B, 3 * _H), _BF),
            pltpu.VMEM((_B, _H), f32),
        ],
        compiler_params=pltpu.CompilerParams(
            dimension_semantics=("arbitrary",)),
    )(outf0, outb0, outf0, outb0, w1f, b1f, w1b, b1b, wfc, bfc)
    return logits


def kernel(input_ids, table,
           W_ih_0_fwd, W_hh_0_fwd, b_ih_0_fwd, b_hh_0_fwd,
           W_ih_0_bwd, W_hh_0_bwd, b_ih_0_bwd, b_hh_0_bwd,
           W_ih_1_fwd, W_hh_1_fwd, b_ih_1_fwd, b_hh_1_fwd,
           W_ih_1_bwd, W_hh_1_bwd, b_ih_1_bwd, b_hh_1_bwd,
           W_fc, b_fc):
    p = dict(locals())
    input_ids = p.pop("input_ids")
    idx_flat = input_ids.T.reshape(_N).astype(jnp.int32)
    x_flat = _emb_gather(p["table"], idx_flat)
    x = x_flat.reshape(_T, _B, _E)
    return _lstm_stack(x, p)


# speedup vs baseline: 4.6835x; 1.0040x over previous
import functools

import jax
import jax.numpy as jnp
import numpy as np
from jax import lax
from jax.experimental import pallas as pl
from jax.experimental.pallas import tpu as pltpu
from jax.experimental.pallas import tpu_sc as plsc

_B, _T = 1024, 200
_V, _E, _H, _C = 30522, 128, 256, 2
_N = _B * _T
_S = 5
_CH = 128
_BF = jnp.bfloat16


def _emb_gather(table, idx_flat):
    info = plsc.get_sparse_core_info()
    nc, ns = info.num_cores, info.num_subcores
    nw = nc * ns
    rows_per_w = _N // nw
    n_ch = rows_per_w // _CH
    idx3d = idx_flat.reshape(nw, n_ch, _CH)
    mesh = plsc.VectorSubcoreMesh(core_axis_name="c", subcore_axis_name="s")

    @functools.partial(
        pl.kernel,
        mesh=mesh,
        out_type=jax.ShapeDtypeStruct((_N, _E), jnp.float32),
        scratch_types=[
            pltpu.VMEM((n_ch, _CH), jnp.int32),
            pltpu.VMEM((_CH, _E), jnp.float32),
            pltpu.VMEM((_CH, _E), jnp.float32),
            pltpu.SemaphoreType.DMA,
            pltpu.SemaphoreType.DMA,
        ],
    )
    def gather_kernel(table_hbm, idx_hbm, out_hbm, idx_v, buf0, buf1, sem0, sem1):
        wid = lax.axis_index("s") * nc + lax.axis_index("c")
        base_row = wid * rows_per_w
        pltpu.sync_copy(idx_hbm.at[wid], idx_v)

        pltpu.async_copy(table_hbm.at[idx_v.at[0]], buf0, sem0)

        def body(j2, carry):
            j = 2 * j2
            pltpu.async_copy(table_hbm.at[idx_v.at[j + 1]], buf1, sem1)
            pltpu.make_async_copy(
                table_hbm.at[idx_v.at[j]], buf0, sem0).wait()
            pltpu.sync_copy(buf0, out_hbm.at[pl.ds(base_row + j * _CH, _CH)])

            @pl.when(j2 < n_ch // 2 - 1)
            def _fire_next():
                pltpu.async_copy(table_hbm.at[idx_v.at[j + 2]], buf0, sem0)

            pltpu.make_async_copy(
                table_hbm.at[idx_v.at[j + 1]], buf1, sem1).wait()
            pltpu.sync_copy(
                buf1, out_hbm.at[pl.ds(base_row + (j + 1) * _CH, _CH)])
            return carry

        lax.fori_loop(0, n_ch // 2, body, 0)

    return gather_kernel(table, idx3d)


def _gate_step(xh_s, w, b, c_s, x_off):
    gates = jnp.dot(xh_s[...], w[...],
                    preferred_element_type=jnp.float32) + b[...]
    i_g = 0.5 * jnp.tanh(gates[:, 0 * _H:1 * _H]) + 0.5
    f_g = 0.5 * jnp.tanh(gates[:, 1 * _H:2 * _H]) + 0.5
    g_g = jnp.tanh(gates[:, 2 * _H:3 * _H])
    o_g = 0.5 * jnp.tanh(gates[:, 3 * _H:4 * _H]) + 0.5
    c_new = f_g * c_s[...] + i_g * g_g
    h_new = o_g * jnp.tanh(c_new)
    c_s[...] = c_new
    h_bf = h_new.astype(_BF)
    xh_s[:, x_off:] = h_bf
    return h_new, h_bf


def _lstm0_body(xf_ref, xb_ref, wf, bf, wb, bb,
                outf, outb, xhf, cf, xhb, cb):
    t = pl.program_id(0)

    @pl.when(t == 0)
    def _init():
        cf[...] = jnp.zeros_like(cf[...])
        cb[...] = jnp.zeros_like(cb[...])
        xhf[:, _E:] = jnp.zeros((_B, _H), _BF)
        xhb[:, _E:] = jnp.zeros((_B, _H), _BF)

    for s in range(_S):
        xhf[:, :_E] = xf_ref[s].astype(_BF)
        xhb[:, :_E] = xb_ref[_S - 1 - s].astype(_BF)
        _, hf_bf = _gate_step(xhf, wf, bf, cf, _E)
        _, hb_bf = _gate_step(xhb, wb, bb, cb, _E)
        outf[s] = hf_bf
        outb[_S - 1 - s] = hb_bf


def _lstm1_body(oft, obt, ofr, obr, wf, bf, wb, bb, wfc, bfc,
                out, xhf, cf, xhb, cb):
    t = pl.program_id(0)

    @pl.when(t == 0)
    def _init():
        cf[...] = jnp.zeros_like(cf[...])
        cb[...] = jnp.zeros_like(cb[...])
        xhf[:, 2 * _H:] = jnp.zeros((_B, _H), _BF)
        xhb[:, 2 * _H:] = jnp.zeros((_B, _H), _BF)

    for s in range(_S):
        xhf[:, 0 * _H:1 * _H] = oft[s]
        xhf[:, 1 * _H:2 * _H] = obt[s]
        xhb[:, 0 * _H:1 * _H] = ofr[_S - 1 - s]
        xhb[:, 1 * _H:2 * _H] = obr[_S - 1 - s]
        h_f, _ = _gate_step(xhf, wf, bf, cf, 2 * _H)
        h_b, _ = _gate_step(xhb, wb, bb, cb, 2 * _H)

    @pl.when(t == _T // _S - 1)
    def _fc():
        w = wfc[...]
        out[...] = (jnp.dot(h_f, w[:_H], preferred_element_type=jnp.float32)
                    + jnp.dot(h_b, w[_H:], preferred_element_type=jnp.float32)
                    + bfc[...])


def _full_spec(shape):
    nd = len(shape)
    return pl.BlockSpec(shape, lambda t, _nd=nd: (0,) * _nd)


_GATE_SCALE = np.concatenate([
    np.full((_H,), 0.5, np.float32),
    np.full((_H,), 0.5, np.float32),
    np.ones((_H,), np.float32),
    np.full((_H,), 0.5, np.float32),
])


def _lstm_stack(x, p):
    f32 = jnp.float32

    wf0 = (jnp.concatenate([p["W_ih_0_fwd"].T, p["W_hh_0_fwd"].T])
           * _GATE_SCALE).astype(_BF)
    wb0 = (jnp.concatenate([p["W_ih_0_bwd"].T, p["W_hh_0_bwd"].T])
           * _GATE_SCALE).astype(_BF)
    bf0 = ((p["b_ih_0_fwd"] + p["b_hh_0_fwd"]) * _GATE_SCALE).reshape(1, 4 * _H)
    bb0 = ((p["b_ih_0_bwd"] + p["b_hh_0_bwd"]) * _GATE_SCALE).reshape(1, 4 * _H)

    outf0, outb0 = pl.pallas_call(
        _lstm0_body,
        grid=(_T // _S,),
        in_specs=[
            pl.BlockSpec((_S, _B, _E), lambda t: (t, 0, 0)),
            pl.BlockSpec((_S, _B, _E), lambda t: (_T // _S - 1 - t, 0, 0)),
            _full_spec((_E + _H, 4 * _H)),
            _full_spec((1, 4 * _H)),
            _full_spec((_E + _H, 4 * _H)),
            _full_spec((1, 4 * _H)),
        ],
        out_specs=[
            pl.BlockSpec((_S, _B, _H), lambda t: (t, 0, 0)),
            pl.BlockSpec((_S, _B, _H), lambda t: (_T // _S - 1 - t, 0, 0)),
        ],
        out_shape=[
            jax.ShapeDtypeStruct((_T, _B, _H), _BF),
            jax.ShapeDtypeStruct((_T, _B, _H), _BF),
        ],
        scratch_shapes=[
            pltpu.VMEM((_B, _E + _H), _BF),
            pltpu.VMEM((_B, _H), f32),
            pltpu.VMEM((_B, _E + _H), _BF),
            pltpu.VMEM((_B, _H), f32),
        ],
        compiler_params=pltpu.CompilerParams(
            dimension_semantics=("arbitrary",)),
    )(x, x, wf0, bf0, wb0, bb0)

    w1f = (jnp.concatenate([p["W_ih_1_fwd"].T, p["W_hh_1_fwd"].T])
           * _GATE_SCALE).astype(_BF)
    w1b = (jnp.concatenate([p["W_ih_1_bwd"].T, p["W_hh_1_bwd"].T])
           * _GATE_SCALE).astype(_BF)
    b1f = ((p["b_ih_1_fwd"] + p["b_hh_1_fwd"]) * _GATE_SCALE).reshape(1, 4 * _H)
    b1b = ((p["b_ih_1_bwd"] + p["b_hh_1_bwd"]) * _GATE_SCALE).reshape(1, 4 * _H)
    wfc = p["W_fc"].T
    bfc = p["b_fc"].reshape(1, _C)

    logits = pl.pallas_call(
        _lstm1_body,
        grid=(_T // _S,),
        in_specs=[
            pl.BlockSpec((_S, _B, _H), lambda t: (t, 0, 0)),
            pl.BlockSpec((_S, _B, _H), lambda t: (t, 0, 0)),
            pl.BlockSpec((_S, _B, _H), lambda t: (_T // _S - 1 - t, 0, 0)),
            pl.BlockSpec((_S, _B, _H), lambda t: (_T // _S - 1 - t, 0, 0)),
            _full_spec((3 * _H, 4 * _H)),
            _full_spec((1, 4 * _H)),
            _full_spec((3 * _H, 4 * _H)),
            _full_spec((1, 4 * _H)),
            _full_spec((2 * _H, _C)),
            _full_spec((1, _C)),
        ],
        out_specs=pl.BlockSpec((_B, _C), lambda t: (0, 0)),
        out_shape=jax.ShapeDtypeStruct((_B, _C), f32),
        scratch_shapes=[
            pltpu.VMEM((_B, 3 * _H), _BF),
            pltpu.VMEM((_B, _H), f32),
            pltpu.VMEM((_B, 3 * _H), _BF),
            pltpu.VMEM((_B, _H), f32),
        ],
        compiler_params=pltpu.CompilerParams(
            dimension_semantics=("arbitrary",)),
    )(outf0, outb0, outf0, outb0, w1f, b1f, w1b, b1b, wfc, bfc)
    return logits


def kernel(input_ids, table,
           W_ih_0_fwd, W_hh_0_fwd, b_ih_0_fwd, b_hh_0_fwd,
           W_ih_0_bwd, W_hh_0_bwd, b_ih_0_bwd, b_hh_0_bwd,
           W_ih_1_fwd, W_hh_1_fwd, b_ih_1_fwd, b_hh_1_fwd,
           W_ih_1_bwd, W_hh_1_bwd, b_ih_1_bwd, b_hh_1_bwd,
           W_fc, b_fc):
    p = dict(locals())
    input_ids = p.pop("input_ids")
    idx_flat = input_ids.T.reshape(_N).astype(jnp.int32)
    x_flat = _emb_gather(p["table"], idx_flat)
    x = x_flat.reshape(_T, _B, _E)
    return _lstm_stack(x, p)
